# baseline (speedup 1.0000x reference)
import jax
import jax.numpy as jnp
from jax.experimental import pallas as pl
from jax.experimental.pallas import tpu as pltpu

_CHUNK = 10000
_NCHUNK = 10


def _copy_body(ent_in, rel_in, ent_out, rel_out, buf, rel_buf,
               in_sems, out_sems, rel_in_sem, rel_out_sem):
    rel_c_in = pltpu.make_async_copy(rel_in, rel_buf, rel_in_sem)
    rel_c_in.start()
    in_copies = []
    for i in range(_NCHUNK):
        c = pltpu.make_async_copy(
            ent_in.at[pl.ds(i * _CHUNK, _CHUNK)], buf.at[i], in_sems.at[i])
        c.start()
        in_copies.append(c)
    rel_c_in.wait()
    rel_c_out = pltpu.make_async_copy(rel_buf, rel_out, rel_out_sem)
    rel_c_out.start()
    out_copies = []
    for i in range(_NCHUNK):
        in_copies[i].wait()
        c = pltpu.make_async_copy(
            buf.at[i], ent_out.at[pl.ds(i * _CHUNK, _CHUNK)], out_sems.at[i])
        c.start()
        out_copies.append(c)
    rel_c_out.wait()
    for c in out_copies:
        c.wait()


def kernel(entity_weight, relation_weight):
    n_ent, d = entity_weight.shape
    ent_out, rel_out = pl.pallas_call(
        _copy_body,
        in_specs=[
            pl.BlockSpec(memory_space=pl.ANY),
            pl.BlockSpec(memory_space=pl.ANY),
        ],
        out_specs=[
            pl.BlockSpec(memory_space=pl.ANY),
            pl.BlockSpec(memory_space=pl.ANY),
        ],
        out_shape=[
            jax.ShapeDtypeStruct(entity_weight.shape, entity_weight.dtype),
            jax.ShapeDtypeStruct(relation_weight.shape, relation_weight.dtype),
        ],
        scratch_shapes=[
            pltpu.VMEM((_NCHUNK, _CHUNK, d), entity_weight.dtype),
            pltpu.VMEM(relation_weight.shape, relation_weight.dtype),
            pltpu.SemaphoreType.DMA((_NCHUNK,)),
            pltpu.SemaphoreType.DMA((_NCHUNK,)),
            pltpu.SemaphoreType.DMA,
            pltpu.SemaphoreType.DMA,
        ],
    )(entity_weight, relation_weight)
    return (ent_out, rel_out)
